# baseline (speedup 1.0000x reference)
import jax
import jax.numpy as jnp
from jax.experimental import pallas as pl
from jax.experimental.pallas import tpu as pltpu

_R = 64
_TS = 256
_OJ = 512


def _cast_body(w_ref, o_ref):
    o_ref[...] = w_ref[...].astype(jnp.bfloat16)


def _main_body(ids_ref, scale_ref, rank_ref, x_ref, w_ref, a_ref, b_ref,
               o_ref):
    s = pl.program_id(0)
    xbf = x_ref[...].astype(jnp.bfloat16)
    xa = jax.lax.dot_general(
        xbf, a_ref[0],
        dimension_numbers=(((1,), (1,)), ((), ())),
        preferred_element_type=jnp.float32)
    col = jax.lax.broadcasted_iota(jnp.int32, (1, _R), 1)
    mask = (col < rank_ref[s]).astype(jnp.float32)
    xa_bf = (xa * (mask * scale_ref[s])).astype(jnp.bfloat16)
    base = jax.lax.dot_general(
        xbf, w_ref[...],
        dimension_numbers=(((1,), (1,)), ((), ())),
        preferred_element_type=jnp.float32)
    lora = jax.lax.dot_general(
        xa_bf, b_ref[0],
        dimension_numbers=(((1,), (0,)), ((), ())),
        preferred_element_type=jnp.float32)
    o_ref[...] = base + lora


def kernel(x, a_cache, b_cache, base_weight, b_adapter_ids, b_scaling, ranks):
    T, D = x.shape
    O = base_weight.shape[0]
    n_s = T // _TS
    seq_len = T // b_adapter_ids.shape[0]

    w_bf = pl.pallas_call(
        _cast_body,
        grid=(O // _OJ,),
        in_specs=[pl.BlockSpec((_OJ, D), lambda j: (j, 0))],
        out_specs=pl.BlockSpec((_OJ, D), lambda j: (j, 0)),
        out_shape=jax.ShapeDtypeStruct((O, D), jnp.bfloat16),
    )(base_weight)
    a_bf = a_cache.astype(jnp.bfloat16)
    b_bf = b_cache.astype(jnp.bfloat16)

    blk_seq = (jnp.arange(n_s, dtype=jnp.int32) * _TS) // seq_len
    ids_blk = b_adapter_ids[blk_seq].astype(jnp.int32)
    scale_blk = b_scaling[blk_seq].astype(jnp.float32)
    rank_blk = ranks[b_adapter_ids][blk_seq].astype(jnp.int32)

    return pl.pallas_call(
        _main_body,
        grid_spec=pltpu.PrefetchScalarGridSpec(
            num_scalar_prefetch=3,
            grid=(n_s,),
            in_specs=[
                pl.BlockSpec((_TS, D), lambda s, ids, sc, rk: (s, 0)),
                pl.BlockSpec((O, D), lambda s, ids, sc, rk: (0, 0)),
                pl.BlockSpec((1, _R, D), lambda s, ids, sc, rk: (ids[s], 0, 0)),
                pl.BlockSpec((1, _R, O), lambda s, ids, sc, rk: (ids[s], 0, 0)),
            ],
            out_specs=pl.BlockSpec((_TS, O), lambda s, ids, sc, rk: (s, 0)),
        ),
        out_shape=jax.ShapeDtypeStruct((T, O), jnp.float32),
        compiler_params=pltpu.CompilerParams(
            dimension_semantics=("arbitrary",)),
    )(ids_blk, scale_blk, rank_blk, x, w_bf, a_bf, b_bf)


# speedup vs baseline: 1.1071x; 1.1071x over previous
import jax
import jax.numpy as jnp
from jax.experimental import pallas as pl
from jax.experimental.pallas import tpu as pltpu

_R = 64
_TS = 1024
_OJ = 512


def _lora_body(ids_ref, scale_ref, rank_ref, x_ref, w_ref, a_ref, b_ref,
               o_ref, xa_ref):
    s = pl.program_id(0)
    j = pl.program_id(1)

    @pl.when(j == 0)
    def _():
        xa = jax.lax.dot_general(
            x_ref[...], a_ref[0],
            dimension_numbers=(((1,), (1,)), ((), ())),
            preferred_element_type=jnp.float32)
        col = jax.lax.broadcasted_iota(jnp.int32, (1, _R), 1)
        mask = (col < rank_ref[s]).astype(jnp.float32)
        xa_ref[...] = xa * (mask * scale_ref[s])

    base = jax.lax.dot_general(
        x_ref[...], w_ref[...],
        dimension_numbers=(((1,), (1,)), ((), ())),
        preferred_element_type=jnp.float32)
    lora = jax.lax.dot_general(
        xa_ref[...], b_ref[0],
        dimension_numbers=(((1,), (0,)), ((), ())),
        preferred_element_type=jnp.float32)
    o_ref[...] = base + lora


def kernel(x, a_cache, b_cache, base_weight, b_adapter_ids, b_scaling, ranks):
    T, D = x.shape
    O = base_weight.shape[0]
    n_s = T // _TS
    n_j = O // _OJ
    seq_len = T // b_adapter_ids.shape[0]

    blk_seq = (jnp.arange(n_s, dtype=jnp.int32) * _TS) // seq_len
    ids_blk = b_adapter_ids[blk_seq].astype(jnp.int32)
    scale_blk = b_scaling[blk_seq].astype(jnp.float32)
    rank_blk = ranks[b_adapter_ids][blk_seq].astype(jnp.int32)

    grid_spec = pltpu.PrefetchScalarGridSpec(
        num_scalar_prefetch=3,
        grid=(n_s, n_j),
        in_specs=[
            pl.BlockSpec((_TS, D), lambda s, j, ids, sc, rk: (s, 0)),
            pl.BlockSpec((_OJ, D), lambda s, j, ids, sc, rk: (j, 0)),
            pl.BlockSpec((1, _R, D), lambda s, j, ids, sc, rk: (ids[s], 0, 0)),
            pl.BlockSpec((1, _R, _OJ), lambda s, j, ids, sc, rk: (ids[s], 0, j)),
        ],
        out_specs=pl.BlockSpec((_TS, _OJ), lambda s, j, ids, sc, rk: (s, j)),
        scratch_shapes=[pltpu.VMEM((_TS, _R), jnp.float32)],
    )
    return pl.pallas_call(
        _lora_body,
        grid_spec=grid_spec,
        out_shape=jax.ShapeDtypeStruct((T, O), jnp.float32),
        compiler_params=pltpu.CompilerParams(
            dimension_semantics=("arbitrary", "arbitrary")),
    )(ids_blk, scale_blk, rank_blk, x, base_weight, a_cache, b_cache)
